# baseline (speedup 1.0000x reference)
import functools

import jax
import jax.numpy as jnp
from jax import lax
from jax.experimental import pallas as pl
from jax.experimental.pallas import tpu as pltpu
from jax.experimental.pallas import tpu_sc as plsc

_D = 64
_W = 128
_G = 128
_K = 2
_NC = 2
_NS = 16
_NW = _NC * _NS


@functools.cache
def _build(n_rows):
    ng = n_rows // (_NW * _G)
    nr = ng // _K
    bw = ng * _G
    blk = _K * _G
    mesh = plsc.VectorSubcoreMesh(core_axis_name="c", subcore_axis_name="s",
                                  num_cores=_NC, num_subcores=_NS)

    @functools.partial(
        pl.kernel,
        out_type=jax.ShapeDtypeStruct((n_rows, _W), jnp.float32),
        mesh=mesh,
        scratch_types=[
            pltpu.VMEM((ng, _G), jnp.int32),
            pltpu.VMEM((blk, _W), jnp.float32),
            pltpu.VMEM((blk, _W), jnp.float32),
            pltpu.SemaphoreType.DMA,
            pltpu.SemaphoreType.DMA,
            pltpu.SemaphoreType.DMA,
            pltpu.SemaphoreType.DMA,
        ],
    )
    def gather_kernel(tokens_hbm, table_hbm, out_hbm,
                      idx_v, buf0, buf1, gsem0, gsem1, osem0, osem1):
        wid = lax.axis_index("s") * _NC + lax.axis_index("c")
        pltpu.sync_copy(tokens_hbm.at[pl.ds(wid * ng, ng)], idx_v)
        base = wid * bw

        def fire(r, buf, gsem):
            for k in range(_K):
                pltpu.async_copy(table_hbm.at[idx_v.at[r * _K + k]],
                                 buf.at[pl.ds(k * _G, _G)], gsem)

        def drain(buf, gsem):
            pltpu.make_async_copy(table_hbm.at[pl.ds(0, blk)], buf, gsem).wait()

        def wb_wait(buf, osem):
            pltpu.make_async_copy(table_hbm.at[pl.ds(0, blk)], buf, osem).wait()

        fire(0, buf0, gsem0)
        fire(1, buf1, gsem1)

        @pl.loop(0, nr, step=2)
        def _round(g):
            off0 = pl.multiple_of(base + g * blk, blk)
            off1 = pl.multiple_of(base + (g + 1) * blk, blk)
            drain(buf0, gsem0)
            pltpu.async_copy(buf0, out_hbm.at[pl.ds(off0, blk)], osem0)
            drain(buf1, gsem1)
            pltpu.async_copy(buf1, out_hbm.at[pl.ds(off1, blk)], osem1)

            @pl.when(g + 2 < nr)
            def _():
                wb_wait(buf0, osem0)
                fire(g + 2, buf0, gsem0)

            @pl.when(g + 3 < nr)
            def _():
                wb_wait(buf1, osem1)
                fire(g + 3, buf1, gsem1)

        wb_wait(buf0, osem0)
        wb_wait(buf1, osem1)

    return gather_kernel


def _widen_body(t_ref, o_ref):
    x = t_ref[...]
    o_ref[...] = jnp.concatenate([x, x], axis=1)


@functools.cache
def _widen_call(v, blk=2000):
    return pl.pallas_call(
        _widen_body,
        grid=(v // blk,),
        in_specs=[pl.BlockSpec((blk, _D), lambda i: (i, 0))],
        out_specs=pl.BlockSpec((blk, _W), lambda i: (i, 0)),
        out_shape=jax.ShapeDtypeStruct((v, _W), jnp.float32),
    )


def _widen(table):
    return _widen_call(table.shape[0])(table)


def kernel(tokens, table):
    B, L = tokens.shape
    n_rows = B * L
    twide = _widen(table)
    flat = tokens.astype(jnp.int32).reshape(n_rows // _G, _G)
    out_wide = _build(n_rows)(flat, twide)
    return out_wide[:, :_D].reshape(B, L, _D)


# speedup vs baseline: 1.3452x; 1.3452x over previous
import functools

import jax
import jax.numpy as jnp
from jax import lax
from jax.experimental import pallas as pl
from jax.experimental.pallas import tpu as pltpu
from jax.experimental.pallas import tpu_sc as plsc

_D = 64
_W = 128
_G = 128
_NB = 5
_NC = 2
_NS = 16
_NW = _NC * _NS


@functools.cache
def _build(n_rows):
    ng = n_rows // (_NW * _G)
    bw = ng * _G
    mesh = plsc.VectorSubcoreMesh(core_axis_name="c", subcore_axis_name="s",
                                  num_cores=_NC, num_subcores=_NS)

    @functools.partial(
        pl.kernel,
        out_type=jax.ShapeDtypeStruct((n_rows, _W), jnp.float32),
        mesh=mesh,
        scratch_types=[
            pltpu.VMEM((ng, _G), jnp.int32),
            [pltpu.VMEM((_G, _W), jnp.float32) for _ in range(_NB)],
            [pltpu.SemaphoreType.DMA for _ in range(_NB)],
            [pltpu.SemaphoreType.DMA for _ in range(_NB)],
        ],
    )
    def gather_kernel(tokens_hbm, table_hbm, out_hbm, idx_v, bufs, gsems, osems):
        wid = lax.axis_index("s") * _NC + lax.axis_index("c")
        pltpu.sync_copy(tokens_hbm.at[pl.ds(wid * ng, ng)], idx_v)
        base = wid * bw

        def fire(r, s):
            pltpu.async_copy(table_hbm.at[idx_v.at[r]], bufs[s], gsems[s])

        def drain(s, sem_list):
            pltpu.make_async_copy(table_hbm.at[pl.ds(0, _G)], bufs[s],
                                  sem_list[s]).wait()

        for s in range(_NB):
            fire(s, s)

        @pl.loop(0, ng, step=_NB)
        def _round(g):
            for s in range(_NB):
                drain(s, gsems)
                off = pl.multiple_of(base + (g + s) * _G, _G)
                pltpu.async_copy(bufs[s], out_hbm.at[pl.ds(off, _G)], osems[s])
            for s in range(_NB):
                @pl.when(g + _NB + s < ng)
                def _(s=s):
                    drain(s, osems)
                    fire(g + _NB + s, s)

        for s in range(_NB):
            drain(s, osems)

    return gather_kernel


def kernel(tokens, table):
    B, L = tokens.shape
    n_rows = B * L
    twide = jnp.pad(table, ((0, 0), (0, _W - table.shape[1])))
    flat = tokens.astype(jnp.int32).reshape(n_rows // _G, _G)
    out_wide = _build(n_rows)(flat, twide)
    return out_wide[:, :_D].reshape(B, L, _D)
